# baseline (speedup 1.0000x reference)
import functools

import jax
import jax.numpy as jnp
from jax.experimental import pallas as pl
from jax.experimental.pallas import tpu as pltpu

_INTERPRET = False


def _bf16(t):
    return t.astype(jnp.bfloat16)


def _pick(total, want):
    b = min(want, total)
    while total % b or b % 8:
        b -= 1
    return b


def _body(kb, bk, mb, bm, h2,
          x_ref, adj_ref, w1_ref, w23_ref, wa1_ref, wa2_ref, wa3_ref,
          mu_ref, lv_ref, px_ref, mua_ref, lva_ref, pa_ref,
          xw1_ref, hw_ref, acc_ref, mub_ref):
    g = pl.program_id(0)

    @pl.when(g < kb)
    def _():
        xb = _bf16(x_ref[...])
        xw1_ref[pl.ds(g * bk, bk), :] = jnp.dot(
            xb, _bf16(w1_ref[...]),
            preferred_element_type=jnp.float32).astype(jnp.bfloat16)
        part = jax.lax.dot_general(xb, _bf16(wa1_ref[...]),
                                   (((0,), (0,)), ((), ())),
                                   preferred_element_type=jnp.float32)

        @pl.when(g == 0)
        def _():
            acc_ref[...] = part

        @pl.when(g != 0)
        def _():
            acc_ref[...] += part

        @pl.when(g == kb - 1)
        def _():
            ha1_b = _bf16(jnp.tanh(acc_ref[...]))
            mua_ref[...] = jnp.dot(ha1_b, _bf16(wa2_ref[...]),
                                   preferred_element_type=jnp.float32)
            lva_ref[...] = jnp.dot(ha1_b, _bf16(wa3_ref[...]),
                                   preferred_element_type=jnp.float32)

    @pl.when((g >= kb) & (g < kb + mb))
    def _():
        part = jnp.dot(_bf16(adj_ref[...]), xw1_ref[...],
                       preferred_element_type=jnp.float32)
        h1b = _bf16(jnp.maximum(part, 0.0))
        hw_ref[pl.ds((g - kb) * bm, bm), :] = jnp.dot(
            h1b, _bf16(w23_ref[...]),
            preferred_element_type=jnp.float32).astype(jnp.bfloat16)

    @pl.when((g >= kb + mb) & (g < kb + 2 * mb))
    def _():
        acc = jnp.dot(_bf16(adj_ref[...]), hw_ref[...],
                      preferred_element_type=jnp.float32)
        mu = acc[:, :h2]
        mu_ref[...] = mu
        lv_ref[...] = acc[:, h2:]
        mu_b = _bf16(mu)
        mub_ref[pl.ds((g - kb - mb) * bm, bm), :] = mu_b
        px_ref[...] = jax.lax.dot_general(
            mu_b, _bf16(mua_ref[...]),
            (((1,), (1,)), ((), ())),
            preferred_element_type=jnp.float32)

    @pl.when(g >= kb + 2 * mb)
    def _():
        i = g - kb - 2 * mb
        pa_ref[...] = jax.lax.dot_general(
            mub_ref[pl.ds(i * bm, bm), :], mub_ref[...],
            (((1,), (1,)), ((), ())),
            preferred_element_type=jnp.float32)


def _main(x, adj, w1, w23, wa1, wa2, wa3):
    n, feat = x.shape
    h1 = w1.shape[1]
    h2 = wa2.shape[1]
    bk = _pick(n, 1000)
    kb = n // bk
    bm = _pick(n, 200)
    mb = n // bm

    def adj_idx(g):
        return (jnp.where(g < kb + mb,
                          jnp.maximum(g - kb, 0),
                          jnp.where(g < kb + 2 * mb, g - kb - mb, mb - 1)), 0)

    def x_idx(g):
        return (jnp.minimum(g, kb - 1), 0)

    def out_idx(g):
        return (jnp.clip(g - kb - mb, 0, mb - 1), 0)

    def pa_idx(g):
        return (jnp.maximum(g - kb - 2 * mb, 0), 0)

    const_idx = lambda g: (0, 0)

    return pl.pallas_call(
        functools.partial(_body, kb, bk, mb, bm, h2),
        grid=(kb + 3 * mb,),
        in_specs=[
            pl.BlockSpec((bk, feat), x_idx),
            pl.BlockSpec((bm, n), adj_idx),
            pl.BlockSpec((feat, h1), const_idx),
            pl.BlockSpec((h1, h1), const_idx),
            pl.BlockSpec((bk, h1), x_idx),
            pl.BlockSpec((h1, h2), const_idx),
            pl.BlockSpec((h1, h2), const_idx),
        ],
        out_specs=[
            pl.BlockSpec((bm, h2), out_idx),
            pl.BlockSpec((bm, h2), out_idx),
            pl.BlockSpec((bm, feat), out_idx),
            pl.BlockSpec((feat, h2), const_idx),
            pl.BlockSpec((feat, h2), const_idx),
            pl.BlockSpec((bm, n), pa_idx),
        ],
        out_shape=[
            jax.ShapeDtypeStruct((n, h2), jnp.float32),
            jax.ShapeDtypeStruct((n, h2), jnp.float32),
            jax.ShapeDtypeStruct((n, feat), jnp.float32),
            jax.ShapeDtypeStruct((feat, h2), jnp.float32),
            jax.ShapeDtypeStruct((feat, h2), jnp.float32),
            jax.ShapeDtypeStruct((n, n), jnp.float32),
        ],
        scratch_shapes=[
            pltpu.VMEM((n, h1), jnp.bfloat16),
            pltpu.VMEM((n, h1), jnp.bfloat16),
            pltpu.VMEM((feat, h1), jnp.float32),
            pltpu.VMEM((n, h2), jnp.bfloat16),
        ],
        interpret=_INTERPRET,
    )(x, adj, w1, w23, wa1, wa2, wa3)


def kernel(x, adj, W1, W2, W3, Wa1, Wa2, Wa3):
    w23 = jnp.concatenate([W2, W3], axis=1)
    mu, logvar, pred_x, mu_a, logvar_a, pred_adj = _main(
        x, adj, W1, w23, Wa1, Wa2, Wa3)
    return (pred_adj, pred_x, mu, logvar, mu_a, logvar_a)


# speedup vs baseline: 1.0263x; 1.0263x over previous
import functools

import jax
import jax.numpy as jnp
from jax.experimental import pallas as pl
from jax.experimental.pallas import tpu as pltpu


def _bf16(t):
    return t.astype(jnp.bfloat16)


def _pick(total, want):
    b = min(want, total)
    while total % b or b % 8:
        b -= 1
    return b


def _main_body(kb, bk, mb, bm, h2,
               x_ref, adj_ref, w1_ref, w23_ref, wa1_ref, wa2_ref, wa3_ref,
               mu_ref, lv_ref, px_ref, mua_ref, lva_ref,
               xw1_ref, hw_ref, acc_ref):
    g = pl.program_id(0)

    @pl.when(g < kb)
    def _():
        xb = _bf16(x_ref[...])
        xw1_ref[pl.ds(g * bk, bk), :] = jnp.dot(
            xb, _bf16(w1_ref[...]),
            preferred_element_type=jnp.float32).astype(jnp.bfloat16)
        part = jax.lax.dot_general(xb, _bf16(wa1_ref[...]),
                                   (((0,), (0,)), ((), ())),
                                   preferred_element_type=jnp.float32)

        @pl.when(g == 0)
        def _():
            acc_ref[...] = part

        @pl.when(g != 0)
        def _():
            acc_ref[...] += part

        @pl.when(g == kb - 1)
        def _():
            ha1_b = _bf16(jnp.tanh(acc_ref[...]))
            mua_ref[...] = jnp.dot(ha1_b, _bf16(wa2_ref[...]),
                                   preferred_element_type=jnp.float32)
            lva_ref[...] = jnp.dot(ha1_b, _bf16(wa3_ref[...]),
                                   preferred_element_type=jnp.float32)

    @pl.when((g >= kb) & (g < kb + mb))
    def _():
        part = jnp.dot(_bf16(adj_ref[...]), xw1_ref[...],
                       preferred_element_type=jnp.float32)
        h1b = _bf16(jnp.maximum(part, 0.0))
        hw_ref[pl.ds((g - kb) * bm, bm), :] = jnp.dot(
            h1b, _bf16(w23_ref[...]),
            preferred_element_type=jnp.float32).astype(jnp.bfloat16)

    @pl.when(g >= kb + mb)
    def _():
        acc = jnp.dot(_bf16(adj_ref[...]), hw_ref[...],
                      preferred_element_type=jnp.float32)
        mu = acc[:, :h2]
        mu_ref[...] = mu
        lv_ref[...] = acc[:, h2:]
        px_ref[...] = jax.lax.dot_general(
            _bf16(mu), _bf16(mua_ref[...]),
            (((1,), (1,)), ((), ())),
            preferred_element_type=jnp.float32)


def _main(x, adj, w1, w23, wa1, wa2, wa3):
    n, feat = x.shape
    h1 = w1.shape[1]
    h2 = wa2.shape[1]
    bk = _pick(n, 2000)
    kb = n // bk
    bm = _pick(n, 400)
    mb = n // bm

    def adj_idx(g):
        return (jnp.where(g < kb + mb,
                          jnp.maximum(g - kb, 0),
                          g - kb - mb), 0)

    def x_idx(g):
        return (jnp.minimum(g, kb - 1), 0)

    def out_idx(g):
        return (jnp.maximum(g - kb - mb, 0), 0)

    const_idx = lambda g: (0, 0)

    return pl.pallas_call(
        functools.partial(_main_body, kb, bk, mb, bm, h2),
        grid=(kb + 2 * mb,),
        in_specs=[
            pl.BlockSpec((bk, feat), x_idx),
            pl.BlockSpec((bm, n), adj_idx),
            pl.BlockSpec((feat, h1), const_idx),
            pl.BlockSpec((h1, h1), const_idx),
            pl.BlockSpec((bk, h1), x_idx),
            pl.BlockSpec((h1, h2), const_idx),
            pl.BlockSpec((h1, h2), const_idx),
        ],
        out_specs=[
            pl.BlockSpec((bm, h2), out_idx),
            pl.BlockSpec((bm, h2), out_idx),
            pl.BlockSpec((bm, feat), out_idx),
            pl.BlockSpec((feat, h2), const_idx),
            pl.BlockSpec((feat, h2), const_idx),
        ],
        out_shape=[
            jax.ShapeDtypeStruct((n, h2), jnp.float32),
            jax.ShapeDtypeStruct((n, h2), jnp.float32),
            jax.ShapeDtypeStruct((n, feat), jnp.float32),
            jax.ShapeDtypeStruct((feat, h2), jnp.float32),
            jax.ShapeDtypeStruct((feat, h2), jnp.float32),
        ],
        scratch_shapes=[
            pltpu.VMEM((n, h1), jnp.bfloat16),
            pltpu.VMEM((n, h1), jnp.bfloat16),
            pltpu.VMEM((feat, h1), jnp.float32),
        ],
    )(x, adj, w1, w23, wa1, wa2, wa3)


def _zzt_body(bm, zall_ref, out_ref, zb_ref):
    i = pl.program_id(0)

    @pl.when(i == 0)
    def _():
        zb_ref[...] = _bf16(zall_ref[...])

    out_ref[...] = jax.lax.dot_general(
        zb_ref[pl.ds(i * bm, bm), :], zb_ref[...],
        (((1,), (1,)), ((), ())),
        preferred_element_type=jnp.float32)


def _zzt(z):
    n, h2 = z.shape
    bm = _pick(n, 400)
    return pl.pallas_call(
        functools.partial(_zzt_body, bm),
        grid=(n // bm,),
        in_specs=[
            pl.BlockSpec((n, h2), lambda i: (0, 0)),
        ],
        out_specs=pl.BlockSpec((bm, n), lambda i: (i, 0)),
        out_shape=jax.ShapeDtypeStruct((n, n), jnp.float32),
        scratch_shapes=[pltpu.VMEM((n, h2), jnp.bfloat16)],
    )(z)


def kernel(x, adj, W1, W2, W3, Wa1, Wa2, Wa3):
    w23 = jnp.concatenate([W2, W3], axis=1)
    mu, logvar, pred_x, mu_a, logvar_a = _main(x, adj, W1, w23, Wa1, Wa2, Wa3)
    pred_adj = _zzt(mu)
    return (pred_adj, pred_x, mu, logvar, mu_a, logvar_a)
